# Initial kernel scaffold; baseline (speedup 1.0000x reference)
import jax
import jax.numpy as jnp
from jax.experimental import pallas as pl


def kernel(feat, etypes, edge_index, weight, w_ih, w_hh, b_ih, b_hh):
    raise NotImplementedError("write your pallas kernel here")


# speedup vs baseline: 14.9618x; 14.9618x over previous
import functools

import jax
import jax.numpy as jnp
from jax import lax
from jax.experimental import pallas as pl
from jax.experimental.pallas import tpu as pltpu
from jax.experimental.pallas import tpu_sc as plsc

N = 10000
E = 320000
T = 4
F = 128
STEPS = 2

NC, NS = 2, 16
NW = NC * NS
CH = 128
K = 79
EPT = K * CH
EPAD = NW * EPT
ROWS = N + 112
RPT = ROWS // NS

def _sc_make_idx_body(et_hbm, src_hbm, idx_hbm, et_v, src_v):
    wid = lax.axis_index("s") * NC + lax.axis_index("c")
    base = wid * EPT
    pltpu.sync_copy(et_hbm.at[pl.ds(base, EPT)], et_v)
    pltpu.sync_copy(src_hbm.at[pl.ds(base, EPT)], src_v)

    def body(i, carry):
        sl = pl.ds(i * 16, 16)
        src_v[sl] = src_v[sl] + et_v[sl] * N
        return carry

    lax.fori_loop(0, EPT // 16, body, 0)
    pltpu.sync_copy(src_v, idx_hbm.at[pl.ds(base, EPT)])


def _sc_segsum_body(table_hbm, idx_hbm, dst_hbm, zeros_hbm, out_hbm,
                    idx_v, dst_v, rows_v, acc_sh, sem):
    cid = lax.axis_index("c")
    sid = lax.axis_index("s")
    wid = sid * NC + cid
    base = wid * EPT
    r0 = sid * RPT
    pltpu.sync_copy(zeros_hbm.at[pl.ds(r0, RPT)], acc_sh.at[pl.ds(r0, RPT)])
    pltpu.sync_copy(idx_hbm.at[pl.ds(base, EPT)], idx_v)
    pltpu.sync_copy(dst_hbm.at[wid], dst_v)
    plsc.subcore_barrier()

    def body(j, carry):
        pltpu.async_copy(
            table_hbm.at[idx_v.at[pl.ds(j * CH, CH)]], rows_v, sem).wait()
        pltpu.sync_copy(rows_v, acc_sh.at[dst_v.at[j]], add=True)
        return carry

    lax.fori_loop(0, K, body, 0)
    plsc.subcore_barrier()
    pltpu.sync_copy(acc_sh.at[pl.ds(r0, RPT)], out_hbm.at[cid, pl.ds(r0, RPT)])


@functools.cache
def _sc_kernels():
    mesh = plsc.VectorSubcoreMesh(core_axis_name="c", subcore_axis_name="s",
                                  num_cores=NC, num_subcores=NS)
    make_idx = pl.kernel(
        _sc_make_idx_body,
        out_type=jax.ShapeDtypeStruct((EPAD,), jnp.int32),
        mesh=mesh,
        scratch_types=[
            pltpu.VMEM((EPT,), jnp.int32),
            pltpu.VMEM((EPT,), jnp.int32),
        ],
    )
    segsum = pl.kernel(
        _sc_segsum_body,
        out_type=jax.ShapeDtypeStruct((NC, ROWS, F), jnp.float32),
        mesh=mesh,
        scratch_types=[
            pltpu.VMEM((EPT,), jnp.int32),
            pltpu.VMEM((K, CH), jnp.int32),
            pltpu.VMEM((CH, F), jnp.float32),
            pltpu.VMEM_SHARED((ROWS, F), jnp.float32),
            pltpu.SemaphoreType.DMA,
        ],
    )
    return make_idx, segsum


_BN = 2000
_NB = N // _BN


def _proj_body(h_ref, w_ref, out_ref):
    out_ref[0] = jnp.dot(h_ref[...], w_ref[0],
                         preferred_element_type=jnp.float32)


_tc_proj = pl.pallas_call(
    _proj_body,
    grid=(T, _NB),
    in_specs=[
        pl.BlockSpec((_BN, F), lambda t, b: (b, 0)),
        pl.BlockSpec((1, F, F), lambda t, b: (t, 0, 0)),
    ],
    out_specs=pl.BlockSpec((1, _BN, F), lambda t, b: (t, b, 0)),
    out_shape=jax.ShapeDtypeStruct((T, N, F), jnp.float32),
)


def _gru_body(p0_ref, p1_ref, h_ref, wih_ref, whh_ref, bih_ref, bhh_ref,
              out_ref):
    a = p0_ref[...] + p1_ref[...]
    h = h_ref[...]
    gi = jnp.dot(a, wih_ref[...], preferred_element_type=jnp.float32) \
        + bih_ref[...]
    gh = jnp.dot(h, whh_ref[...], preferred_element_type=jnp.float32) \
        + bhh_ref[...]
    r = jax.nn.sigmoid(gi[:, 0:F] + gh[:, 0:F])
    z = jax.nn.sigmoid(gi[:, F:2 * F] + gh[:, F:2 * F])
    n = jnp.tanh(gi[:, 2 * F:3 * F] + r * gh[:, 2 * F:3 * F])
    out_ref[...] = (1.0 - z) * n + z * h


_tc_gru = pl.pallas_call(
    _gru_body,
    grid=(_NB,),
    in_specs=[
        pl.BlockSpec((_BN, F), lambda b: (b, 0)),
        pl.BlockSpec((_BN, F), lambda b: (b, 0)),
        pl.BlockSpec((_BN, F), lambda b: (b, 0)),
        pl.BlockSpec((F, 3 * F), lambda b: (0, 0)),
        pl.BlockSpec((F, 3 * F), lambda b: (0, 0)),
        pl.BlockSpec((1, 3 * F), lambda b: (0, 0)),
        pl.BlockSpec((1, 3 * F), lambda b: (0, 0)),
    ],
    out_specs=pl.BlockSpec((_BN, F), lambda b: (b, 0)),
    out_shape=jax.ShapeDtypeStruct((N, F), jnp.float32),
)


def kernel(feat, etypes, edge_index, weight, w_ih, w_hh, b_ih, b_hh):
    h = feat
    W = weight.reshape(T, F, F)
    src = edge_index[0]
    dst = edge_index[1]
    pad = EPAD - E
    et_p = jnp.concatenate([etypes, jnp.zeros((pad,), jnp.int32)])
    src_p = jnp.concatenate([src, jnp.zeros((pad,), jnp.int32)])
    dst_p = jnp.concatenate(
        [dst, jnp.full((pad,), N, jnp.int32)]).reshape(NW, K, CH)
    zeros = jnp.zeros((ROWS, F), jnp.float32)
    wih_t = w_ih.T
    whh_t = w_hh.T
    bih = b_ih.reshape(1, 3 * F)
    bhh = b_hh.reshape(1, 3 * F)

    sc_make_idx, sc_segsum = _sc_kernels()
    idx = sc_make_idx(et_p, src_p)
    for _ in range(STEPS):
        table = _tc_proj(h, W).reshape(T * N, F)
        parts = sc_segsum(table, idx, dst_p, zeros)
        h = _tc_gru(parts[0, :N], parts[1, :N], h, wih_t, whh_t, bih, bhh)
    return h
